# Initial kernel scaffold; baseline (speedup 1.0000x reference)
import jax
import jax.numpy as jnp
from jax.experimental import pallas as pl


def kernel(x, edge_index, W1, W2, W3):
    raise NotImplementedError("write your pallas kernel here")


# speedup vs baseline: 5.5577x; 5.5577x over previous
import functools

import jax
import jax.numpy as jnp
from jax import lax
from jax.experimental import pallas as pl
from jax.experimental.pallas import tpu as pltpu
from jax.experimental.pallas import tpu_sc as plsc

N = 10000
E = 320000
D = 128

NP = 10240
NW = 32
CH = 80
K = 128
EPAD = NW * CH * K
PAD_NODE = 10200
TPB = NP // 16
NSTAGE = TPB // K

_f32 = jnp.float32
_i32 = jnp.int32


def _make_deg():
    mesh = plsc.VectorSubcoreMesh(core_axis_name="c", subcore_axis_name="s")

    @functools.partial(
        pl.kernel,
        mesh=mesh,
        out_type=(
            jax.ShapeDtypeStruct((NP,), _f32),
            jax.ShapeDtypeStruct((NP,), _f32),
        ),
        scratch_types=[
            pltpu.VMEM((K,), _i32),
            pltpu.VMEM((K,), _f32),
            pltpu.VMEM((TPB,), _f32),
            pltpu.VMEM_SHARED((NP,), _f32),
        ],
    )
    def deg_kernel(dstp, deg0, deg1, didx, ones, stag, degsh):
        c = lax.axis_index("c")
        s = lax.axis_index("s")
        wid = s * 2 + c
        base = s * TPB

        zf = jnp.zeros((16,), _f32)
        onef = jnp.ones((16,), _f32)

        def zrow(i, carry):
            stag[pl.ds(i * 16, 16)] = zf
            return carry

        lax.fori_loop(0, TPB // 16, zrow, 0)

        def orow(i, carry):
            ones[pl.ds(i * 16, 16)] = onef
            return carry

        lax.fori_loop(0, K // 16, orow, 0)

        pltpu.sync_copy(stag.at[:], degsh.at[pl.ds(base, TPB)])
        plsc.subcore_barrier()

        def body(j, carry):
            pltpu.sync_copy(dstp.at[wid, j], didx)
            pltpu.sync_copy(ones.at[:], degsh.at[didx], add=True)
            return carry

        lax.fori_loop(0, CH, body, 0)
        plsc.subcore_barrier()

        pltpu.sync_copy(degsh.at[pl.ds(base, TPB)], stag)

        @pl.when(c == 0)
        def _():
            pltpu.sync_copy(stag.at[:], deg0.at[pl.ds(base, TPB)])

        @pl.when(c == 1)
        def _():
            pltpu.sync_copy(stag.at[:], deg1.at[pl.ds(base, TPB)])

    return deg_kernel


def _make_agg():
    mesh = plsc.VectorSubcoreMesh(core_axis_name="c", subcore_axis_name="s")

    @functools.partial(
        pl.kernel,
        mesh=mesh,
        out_type=(
            jax.ShapeDtypeStruct((NP, D), _f32),
            jax.ShapeDtypeStruct((NP, D), _f32),
        ),
        scratch_types=[
            pltpu.VMEM((K,), _i32),
            pltpu.VMEM((K,), _i32),
            pltpu.VMEM((K, D), _f32),
            pltpu.VMEM_SHARED((NP, D), _f32),
            pltpu.SemaphoreType.DMA,
        ],
    )
    def agg_kernel(g, srcp, dstp, out0, out1, sidx, didx, rows, acc, sem):
        c = lax.axis_index("c")
        s = lax.axis_index("s")
        wid = s * 2 + c
        base = s * TPB

        @pl.when(c == 0)
        def _():
            for t in range(NSTAGE):
                r0 = base + t * K
                pltpu.sync_copy(g.at[pl.ds(r0, K)], rows)
                pltpu.sync_copy(rows.at[:], acc.at[pl.ds(r0, K)])

        @pl.when(c == 1)
        def _():
            zf = jnp.zeros((16,), _f32)

            def zrow(i, carry):
                for jj in range(D // 16):
                    rows[i, pl.ds(jj * 16, 16)] = zf
                return carry

            lax.fori_loop(0, K, zrow, 0)
            for t in range(NSTAGE):
                r0 = base + t * K
                pltpu.sync_copy(rows.at[:], acc.at[pl.ds(r0, K)])

        plsc.subcore_barrier()

        def body(j, carry):
            pltpu.sync_copy(srcp.at[wid, j], sidx)
            pltpu.sync_copy(dstp.at[wid, j], didx)
            pltpu.async_copy(g.at[sidx], rows, sem).wait()
            pltpu.sync_copy(rows.at[:], acc.at[didx], add=True)
            return carry

        lax.fori_loop(0, CH, body, 0)
        plsc.subcore_barrier()

        @pl.when(c == 0)
        def _():
            for t in range(NSTAGE):
                r0 = base + t * K
                pltpu.sync_copy(acc.at[pl.ds(r0, K)], rows)
                pltpu.sync_copy(rows.at[:], out0.at[pl.ds(r0, K)])

        @pl.when(c == 1)
        def _():
            for t in range(NSTAGE):
                r0 = base + t * K
                pltpu.sync_copy(acc.at[pl.ds(r0, K)], rows)
                pltpu.sync_copy(rows.at[:], out1.at[pl.ds(r0, K)])

    return agg_kernel


_deg_kernel = _make_deg()
_agg_kernel = _make_agg()


def _sb_body(d0, d1, o):
    srows = 1.0 / jnp.sqrt(1.0 + d0[...] + d1[...])
    lane = lax.broadcasted_iota(_i32, (K, K), 1)
    sub = lax.broadcasted_iota(_i32, (K, K), 0)
    for r in range(8):
        srow = srows[r : r + 1, :]
        sd = jnp.where(lane == sub, jnp.broadcast_to(srow, (K, K)), 0.0)
        o[r * K : (r + 1) * K, :] = jnp.broadcast_to(
            jnp.sum(sd, axis=1, keepdims=True), (K, K)
        )


def _sb(d0, d1):
    return pl.pallas_call(
        _sb_body,
        grid=(NP // (8 * K),),
        in_specs=[
            pl.BlockSpec((8, K), lambda b: (b, 0)),
            pl.BlockSpec((8, K), lambda b: (b, 0)),
        ],
        out_specs=pl.BlockSpec((8 * K, K), lambda b: (b, 0)),
        out_shape=jax.ShapeDtypeStruct((NP, K), _f32),
    )(d0.reshape(NP // K, K), d1.reshape(NP // K, K))


_RB = 512


def _g1_body(x, w, sb, o):
    o[...] = sb[...] * jnp.dot(x[...], w[...], preferred_element_type=_f32)


def _g1(xp, w, sb):
    return pl.pallas_call(
        _g1_body,
        grid=(NP // _RB,),
        in_specs=[
            pl.BlockSpec((_RB, D), lambda b: (b, 0)),
            pl.BlockSpec((D, D), lambda b: (0, 0)),
            pl.BlockSpec((_RB, D), lambda b: (b, 0)),
        ],
        out_specs=pl.BlockSpec((_RB, D), lambda b: (b, 0)),
        out_shape=jax.ShapeDtypeStruct((NP, D), _f32),
    )(xp, w, sb)


def _g23_body(a0, a1, sb, w, o):
    t = jnp.maximum(sb[...] * (a0[...] + a1[...]), 0.0)
    o[...] = sb[...] * jnp.dot(t, w[...], preferred_element_type=_f32)


def _g23(a0, a1, sb, w):
    return pl.pallas_call(
        _g23_body,
        grid=(NP // _RB,),
        in_specs=[
            pl.BlockSpec((_RB, D), lambda b: (b, 0)),
            pl.BlockSpec((_RB, D), lambda b: (b, 0)),
            pl.BlockSpec((_RB, D), lambda b: (b, 0)),
            pl.BlockSpec((D, D), lambda b: (0, 0)),
        ],
        out_specs=pl.BlockSpec((_RB, D), lambda b: (b, 0)),
        out_shape=jax.ShapeDtypeStruct((NP, D), _f32),
    )(a0, a1, sb, w)


def _final_body(a0, a1, sb, o):
    o[...] = sb[...] * (a0[...] + a1[...])


def _final(a0, a1, sb):
    return pl.pallas_call(
        _final_body,
        grid=(NP // _RB,),
        in_specs=[
            pl.BlockSpec((_RB, D), lambda b: (b, 0)),
            pl.BlockSpec((_RB, D), lambda b: (b, 0)),
            pl.BlockSpec((_RB, D), lambda b: (b, 0)),
        ],
        out_specs=pl.BlockSpec((_RB, D), lambda b: (b, 0)),
        out_shape=jax.ShapeDtypeStruct((NP, D), _f32),
    )(a0, a1, sb)


def kernel(x, edge_index, W1, W2, W3):
    src = edge_index[0].astype(_i32)
    dst = edge_index[1].astype(_i32)
    pad = jnp.full((EPAD - E,), PAD_NODE, _i32)
    srcp = jnp.concatenate([src, pad]).reshape(NW, CH, K)
    dstp = jnp.concatenate([dst, pad]).reshape(NW, CH, K)
    xp = jnp.pad(x, ((0, NP - N), (0, 0)))

    d0, d1 = _deg_kernel(dstp)
    sb = _sb(d0, d1)

    g = _g1(xp, W1, sb)
    a0, a1 = _agg_kernel(g, srcp, dstp)
    g = _g23(a0, a1, sb, W2)
    a0, a1 = _agg_kernel(g, srcp, dstp)
    g = _g23(a0, a1, sb, W3)
    a0, a1 = _agg_kernel(g, srcp, dstp)
    return _final(a0, a1, sb)[:N]


# speedup vs baseline: 26.9679x; 4.8523x over previous
import functools

import jax
import jax.numpy as jnp
from jax import lax
from jax.experimental import pallas as pl
from jax.experimental.pallas import tpu as pltpu
from jax.experimental.pallas import tpu_sc as plsc

N = 10000
E = 320000
D = 128

NP = 10240
NW = 32
CH = 80
K = 128
EPAD = NW * CH * K
PAD_NODE = 10016
TPB = NP // 16
NSTAGE = TPB // K

_f32 = jnp.float32
_i32 = jnp.int32


def _make_deg():
    mesh = plsc.VectorSubcoreMesh(core_axis_name="c", subcore_axis_name="s")

    @functools.partial(
        pl.kernel,
        mesh=mesh,
        out_type=(
            jax.ShapeDtypeStruct((NP,), _f32),
            jax.ShapeDtypeStruct((NP,), _f32),
        ),
        scratch_types=[
            pltpu.VMEM((CH, K), _i32),
            pltpu.VMEM((K,), _f32),
            pltpu.VMEM((TPB,), _f32),
            pltpu.VMEM_SHARED((NP,), _f32),
        ],
    )
    def deg_kernel(dstp, deg0, deg1, didx, ones, stag, degsh):
        c = lax.axis_index("c")
        s = lax.axis_index("s")
        wid = s * 2 + c
        base = s * TPB

        zf = jnp.zeros((16,), _f32)
        onef = jnp.ones((16,), _f32)

        def zrow(i, carry):
            stag[pl.ds(i * 16, 16)] = zf
            return carry

        lax.fori_loop(0, TPB // 16, zrow, 0)

        def orow(i, carry):
            ones[pl.ds(i * 16, 16)] = onef
            return carry

        lax.fori_loop(0, K // 16, orow, 0)

        pltpu.sync_copy(stag.at[:], degsh.at[pl.ds(base, TPB)])
        pltpu.sync_copy(dstp.at[wid], didx)
        plsc.subcore_barrier()

        def body(j, carry):
            pltpu.sync_copy(ones.at[:], degsh.at[didx.at[j]], add=True)
            return carry

        lax.fori_loop(0, CH, body, 0)
        plsc.subcore_barrier()

        pltpu.sync_copy(degsh.at[pl.ds(base, TPB)], stag)

        @pl.when(c == 0)
        def _():
            pltpu.sync_copy(stag.at[:], deg0.at[pl.ds(base, TPB)])

        @pl.when(c == 1)
        def _():
            pltpu.sync_copy(stag.at[:], deg1.at[pl.ds(base, TPB)])

    return deg_kernel


def _make_agg():
    mesh = plsc.VectorSubcoreMesh(core_axis_name="c", subcore_axis_name="s")

    @functools.partial(
        pl.kernel,
        mesh=mesh,
        out_type=(
            jax.ShapeDtypeStruct((NP, D), _f32),
            jax.ShapeDtypeStruct((NP, D), _f32),
        ),
        scratch_types=[
            pltpu.VMEM((CH // 2, K), _i32),
            pltpu.VMEM((CH // 2, K), _i32),
            pltpu.VMEM((K, D), _f32),
            pltpu.VMEM((K, D), _f32),
            pltpu.VMEM_SHARED((NP, D), _f32),
            pltpu.SemaphoreType.DMA,
            pltpu.SemaphoreType.DMA,
        ],
    )
    def agg_kernel(g, srcp, dstp, out0, out1, sidx, didx, rows0, rows1,
                   acc, gsem0, gsem1):
        c = lax.axis_index("c")
        s = lax.axis_index("s")
        wid = s * 2 + c
        base = s * TPB
        bufs = ((rows0, gsem0), (rows1, gsem1))
        PH = CH // 2

        @pl.when(c == 0)
        def _():
            pltpu.sync_copy(g.at[pl.ds(base, TPB)], acc.at[pl.ds(base, TPB)])

        @pl.when(c == 1)
        def _():
            zf = jnp.zeros((16,), _f32)

            def zrow(i, carry):
                for jj in range(D // 16):
                    rows0[i, pl.ds(jj * 16, 16)] = zf
                return carry

            lax.fori_loop(0, K, zrow, 0)
            for t in range(NSTAGE):
                r0 = base + t * K
                pltpu.sync_copy(rows0.at[:], acc.at[pl.ds(r0, K)])

        pltpu.sync_copy(srcp.at[wid, pl.ds(0, PH)], sidx)
        pltpu.sync_copy(dstp.at[wid, pl.ds(0, PH)], didx)
        plsc.subcore_barrier()

        for p in range(2):
            pltpu.async_copy(g.at[sidx.at[0]], rows0, gsem0)
            pltpu.async_copy(g.at[sidx.at[1]], rows1, gsem1)

            def outer(jj, carry):
                for b, (rows, gsem) in enumerate(bufs):
                    j = jj * 2 + b
                    pltpu.make_async_copy(g.at[sidx.at[j]], rows, gsem).wait()
                    pltpu.sync_copy(rows.at[:], acc.at[didx.at[j]], add=True)

                    @pl.when(j + 2 < PH)
                    def _():
                        pltpu.async_copy(g.at[sidx.at[j + 2]], rows, gsem)

                return carry

            lax.fori_loop(0, PH // 2, outer, 0)
            if p == 0:
                pltpu.sync_copy(srcp.at[wid, pl.ds(PH, PH)], sidx)
                pltpu.sync_copy(dstp.at[wid, pl.ds(PH, PH)], didx)

        plsc.subcore_barrier()

        @pl.when(c == 0)
        def _():
            pltpu.sync_copy(acc.at[pl.ds(base, TPB)], out0.at[pl.ds(base, TPB)])

        @pl.when(c == 1)
        def _():
            pltpu.sync_copy(acc.at[pl.ds(base, TPB)], out1.at[pl.ds(base, TPB)])

    return agg_kernel


_deg_kernel = _make_deg()
_agg_kernel = _make_agg()


def _sb_body(d0, d1, o):
    srows = 1.0 / jnp.sqrt(1.0 + d0[...] + d1[...])
    lane = lax.broadcasted_iota(_i32, (K, K), 1)
    sub = lax.broadcasted_iota(_i32, (K, K), 0)
    for r in range(8):
        srow = srows[r : r + 1, :]
        sd = jnp.where(lane == sub, jnp.broadcast_to(srow, (K, K)), 0.0)
        o[r * K : (r + 1) * K, :] = jnp.broadcast_to(
            jnp.sum(sd, axis=1, keepdims=True), (K, K)
        )


def _sb(d0, d1):
    return pl.pallas_call(
        _sb_body,
        grid=(NP // (8 * K),),
        in_specs=[
            pl.BlockSpec((8, K), lambda b: (b, 0)),
            pl.BlockSpec((8, K), lambda b: (b, 0)),
        ],
        out_specs=pl.BlockSpec((8 * K, K), lambda b: (b, 0)),
        out_shape=jax.ShapeDtypeStruct((NP, K), _f32),
    )(d0.reshape(NP // K, K), d1.reshape(NP // K, K))


_RB = 512


def _g1_body(x, w, sb, o):
    o[...] = sb[...] * jnp.dot(x[...], w[...], preferred_element_type=_f32)


def _g1(xp, w, sb):
    return pl.pallas_call(
        _g1_body,
        grid=(NP // _RB,),
        in_specs=[
            pl.BlockSpec((_RB, D), lambda b: (b, 0)),
            pl.BlockSpec((D, D), lambda b: (0, 0)),
            pl.BlockSpec((_RB, D), lambda b: (b, 0)),
        ],
        out_specs=pl.BlockSpec((_RB, D), lambda b: (b, 0)),
        out_shape=jax.ShapeDtypeStruct((NP, D), _f32),
    )(xp, w, sb)


def _g23_body(a0, a1, sb, w, o):
    t = jnp.maximum(sb[...] * (a0[...] + a1[...]), 0.0)
    o[...] = sb[...] * jnp.dot(t, w[...], preferred_element_type=_f32)


def _g23(a0, a1, sb, w):
    return pl.pallas_call(
        _g23_body,
        grid=(NP // _RB,),
        in_specs=[
            pl.BlockSpec((_RB, D), lambda b: (b, 0)),
            pl.BlockSpec((_RB, D), lambda b: (b, 0)),
            pl.BlockSpec((_RB, D), lambda b: (b, 0)),
            pl.BlockSpec((D, D), lambda b: (0, 0)),
        ],
        out_specs=pl.BlockSpec((_RB, D), lambda b: (b, 0)),
        out_shape=jax.ShapeDtypeStruct((NP, D), _f32),
    )(a0, a1, sb, w)


def _final_body(a0, a1, sb, o):
    o[...] = sb[...] * (a0[...] + a1[...])


def _final(a0, a1, sb):
    return pl.pallas_call(
        _final_body,
        grid=(NP // _RB,),
        in_specs=[
            pl.BlockSpec((_RB, D), lambda b: (b, 0)),
            pl.BlockSpec((_RB, D), lambda b: (b, 0)),
            pl.BlockSpec((_RB, D), lambda b: (b, 0)),
        ],
        out_specs=pl.BlockSpec((_RB, D), lambda b: (b, 0)),
        out_shape=jax.ShapeDtypeStruct((NP, D), _f32),
    )(a0, a1, sb)


def kernel(x, edge_index, W1, W2, W3):
    src = edge_index[0].astype(_i32)
    dst = edge_index[1].astype(_i32)
    pad = PAD_NODE + (jnp.arange(EPAD - E, dtype=_i32) % (NP - PAD_NODE))
    srcp = jnp.concatenate([src, pad]).reshape(NW, CH, K)
    dstp = jnp.concatenate([dst, pad]).reshape(NW, CH, K)
    xp = jnp.pad(x, ((0, NP - N), (0, 0)))

    d0, d1 = _deg_kernel(dstp)
    sb = _sb(d0, d1)

    g = _g1(xp, W1, sb)
    a0, a1 = _agg_kernel(g, srcp, dstp)
    g = _g23(a0, a1, sb, W2)
    a0, a1 = _agg_kernel(g, srcp, dstp)
    g = _g23(a0, a1, sb, W3)
    a0, a1 = _agg_kernel(g, srcp, dstp)
    return _final(a0, a1, sb)[:N]
